# baseline (speedup 1.0000x reference)
import functools

import jax
import jax.numpy as jnp
from jax import lax
from jax.experimental import pallas as pl
from jax.experimental.pallas import tpu as pltpu
from jax.experimental.pallas import tpu_sc as plsc

_BATCH = 16384
_NPOS = 200
_NC, _NS = 2, 16
_NW = _NC * _NS
_CW = 128
_NG = _CW // 16

_SC_COLS = _NW * _CW
_TC_COLS = _BATCH - _SC_COLS
_CB = 4096
_TC_OFF = _SC_COLS // _CB


def _sc_body(pos_hbm, emb_hbm, out_hbm, emb_v, buf, out_v, sem):
    c = lax.axis_index("c")
    s = lax.axis_index("s")
    wid = s * _NC + c
    base = wid * _CW
    pltpu.async_copy(pos_hbm.at[:, pl.ds(base, _CW)], buf, sem)
    pltpu.sync_copy(emb_hbm, emb_v)
    lane = lax.iota(jnp.int32, 16)
    lanes = [lane + (g * 16) for g in range(_NG)]
    neg_inf = jnp.full((16,), -jnp.inf, jnp.float32)
    zero = jnp.zeros((16,), jnp.int32)
    pltpu.make_async_copy(pos_hbm.at[:, pl.ds(base, _CW)], buf, sem).wait()

    def jstep(j, carry):
        jv = jnp.full((16,), j, jnp.int32)
        out = []
        for g in range(_NG):
            cur, idx = carry[2 * g], carry[2 * g + 1]
            v = plsc.load_gather(buf, [jv, lanes[g]])
            cond = v > cur
            out.append(jnp.where(cond, v, cur))
            out.append(jnp.where(cond, jv, idx))
        return tuple(out)

    init = tuple(x for _ in range(_NG) for x in (neg_inf, zero))
    carry = lax.fori_loop(0, _NPOS, jstep, init, unroll=4)
    for g in range(_NG):
        out_v[pl.ds(g * 16, 16)] = plsc.load_gather(emb_v, [carry[2 * g + 1]])
    pltpu.sync_copy(out_v, out_hbm.at[pl.ds(base, _CW)])


def _sc_argmax_embed(pos_t, embedding):
    return pl.kernel(
        _sc_body,
        out_type=jax.ShapeDtypeStruct((_SC_COLS,), jnp.float32),
        mesh=plsc.VectorSubcoreMesh(
            core_axis_name="c", subcore_axis_name="s",
            num_cores=_NC, num_subcores=_NS),
        compiler_params=pltpu.CompilerParams(
            use_tc_tiling_on_sc=True, needs_layout_passes=False),
        scratch_types=[
            pltpu.VMEM((_NPOS,), jnp.float32),
            pltpu.VMEM((_NPOS, _CW), jnp.float32),
            pltpu.VMEM((_CW,), jnp.float32),
            pltpu.SemaphoreType.DMA,
        ],
    )(pos_t, embedding)


def _tc_body(pos_ref, emb_ref, out_ref):
    pos = pos_ref[...]
    m = jnp.max(pos, axis=0, keepdims=True)
    row = lax.broadcasted_iota(jnp.int32, pos.shape, 0)
    cand = jnp.where(pos == m, row, _NPOS)
    idx = jnp.min(cand, axis=0, keepdims=True)
    emb = emb_ref[...]
    val = jnp.max(jnp.where(row == idx, emb, -jnp.inf), axis=0, keepdims=True)
    out_ref[...] = val


def _tc_argmax_embed(pos_t, embedding):
    emb2d = embedding.reshape(_NPOS, 1)
    out = pl.pallas_call(
        _tc_body,
        grid=(_TC_COLS // _CB,),
        in_specs=[
            pl.BlockSpec((_NPOS, _CB), lambda i: (0, i + _TC_OFF)),
            pl.BlockSpec((_NPOS, 1), lambda i: (0, 0)),
        ],
        out_specs=pl.BlockSpec((1, _CB), lambda i: (0, i)),
        out_shape=jax.ShapeDtypeStruct((1, _TC_COLS), jnp.float32),
    )(pos_t, emb2d)
    return out.reshape(_TC_COLS)


@jax.jit
def kernel(position, embedding):
    pos_t = position.T
    sc_out = _sc_argmax_embed(pos_t, embedding)
    tc_out = _tc_argmax_embed(pos_t, embedding)
    return jnp.concatenate([sc_out, tc_out])
